# baseline (speedup 1.0000x reference)
import functools

import jax
import jax.numpy as jnp
from jax import lax
from jax.experimental import pallas as pl
from jax.experimental.pallas import tpu as pltpu
from jax.experimental.pallas import tpu_sc as plsc

RESID_SCALE = 1.0 / 3.45
CODESZ = 8


def _sc_dims():
    try:
        info = plsc.get_sparse_core_info()
        return info.num_cores, info.num_subcores, info.num_lanes
    except (RuntimeError, ValueError):
        return 2, 16, 16


def _widen_grid(grid, br=8192):
    G, D = grid.shape

    def body(g_ref, o_ref):
        g = g_ref[...]
        o_ref[...] = jnp.concatenate([g, g], axis=1)

    return pl.pallas_call(
        body,
        grid=(G // br,),
        in_specs=[pl.BlockSpec((br, D), lambda i: (i, 0))],
        out_specs=pl.BlockSpec((br, 2 * D), lambda i: (i, 0)),
        out_shape=jax.ShapeDtypeStruct((G, 2 * D), jnp.float32),
    )(grid)


def _make_decompress(n_packed):
    NC, NS, L = _sc_dims()
    NW = NC * NS
    assert n_packed % NW == 0
    per_w = n_packed // NW
    CH = 1024
    assert per_w % (2 * CH) == 0
    n_ch = per_w // CH
    STREAM = 128
    n_streams = CH // STREAM
    mesh = plsc.VectorSubcoreMesh(core_axis_name="c", subcore_axis_name="s",
                                  num_cores=NC, num_subcores=NS)

    @functools.partial(
        pl.kernel,
        out_type=jax.ShapeDtypeStruct((n_packed * CODESZ,), jnp.float32),
        mesh=mesh,
        scratch_types=[
            pltpu.VMEM((2, CH), jnp.int32),
            pltpu.VMEM((2, CH), jnp.int32),
            pltpu.VMEM((2, CH), jnp.int32),
            pltpu.VMEM((2, CH, 2 * CODESZ), jnp.float32),
            pltpu.VMEM((2, CH, 2 * CODESZ), jnp.float32),
            pltpu.VMEM((2, CH * CODESZ), jnp.float32),
            pltpu.SemaphoreType.DMA,
            pltpu.SemaphoreType.DMA,
            pltpu.SemaphoreType.DMA,
            pltpu.SemaphoreType.DMA,
        ],
        compiler_params=pltpu.CompilerParams(use_tc_tiling_on_sc=False,
                                             vmem_limit_bytes=4 * 1024 * 1024),
    )
    def decompress(q_hbm, grid2_hbm, w_hbm, packed_v, hi_v, lo_v, ga_v, gb_v,
                   wbuf_v, gsem0, gsem1, osem0, osem1):
        wid = lax.axis_index("s") * NC + lax.axis_index("c")
        base = wid * per_w
        lane = lax.iota(jnp.int32, L)
        lmask = lane < CODESZ
        sixteen = jnp.full((L,), 16, jnp.int32)
        mask16 = jnp.full((L,), 0xFFFF, jnp.int32)
        scale_v = jnp.full((L,), RESID_SCALE, jnp.float32)

        def fire(c, b, gsem):
            row0 = base + c * CH
            pltpu.sync_copy(q_hbm.at[pl.ds(row0, CH)], packed_v.at[b])

            def split_body(i, _):
                v = packed_v[b, pl.ds(i * L, L)]
                hi_v[b, pl.ds(i * L, L)] = lax.shift_right_logical(v, sixteen)
                lo_v[b, pl.ds(i * L, L)] = lax.bitwise_and(v, mask16)
                return 0

            lax.fori_loop(0, CH // L, split_body, 0, unroll=4)
            for j in range(n_streams):
                sl = pl.ds(j * STREAM, STREAM)
                pltpu.async_copy(
                    grid2_hbm.at[hi_v.at[b, sl]], ga_v.at[b, sl], gsem)
                pltpu.async_copy(
                    grid2_hbm.at[lo_v.at[b, sl]], gb_v.at[b, sl], gsem)

        def wait_gathers(b, gsem):
            dummy = grid2_hbm.at[pl.ds(0, CH)]
            pltpu.make_async_copy(dummy, ga_v.at[b], gsem).wait()
            pltpu.make_async_copy(dummy, gb_v.at[b], gsem).wait()

        def fma_store(c, b, osem):
            @pl.when(c >= 2)
            def _wait_prev_store():
                pltpu.make_async_copy(
                    wbuf_v.at[b], w_hbm.at[pl.ds(base * CODESZ, CH * CODESZ)],
                    osem).wait()

            @plsc.parallel_loop(0, CH // 2, unroll=8)
            def fma_body(i):
                a0 = ga_v[b, 2 * i, :]
                a1 = ga_v[b, 2 * i + 1, :]
                b0 = gb_v[b, 2 * i, :]
                b1 = gb_v[b, 2 * i + 1, :]
                wa = jnp.where(lmask, a0, a1)
                wb = jnp.where(lmask, b0, b1)
                wbuf_v[b, pl.ds(i * L, L)] = wa + wb * scale_v

            row0 = base + c * CH
            pltpu.async_copy(
                wbuf_v.at[b], w_hbm.at[pl.ds(row0 * CODESZ, CH * CODESZ)], osem)

        fire(0, 0, gsem0)

        def loop_body(c2, carry):
            c = 2 * c2

            @pl.when(c + 1 < n_ch)
            def _f1():
                fire(c + 1, 1, gsem1)

            wait_gathers(0, gsem0)
            fma_store(c, 0, osem0)

            @pl.when(c + 2 < n_ch)
            def _f2():
                fire(c + 2, 0, gsem0)

            wait_gathers(1, gsem1)
            fma_store(c + 1, 1, osem1)
            return carry

        lax.fori_loop(0, n_ch // 2, loop_body, 0)
        tail = w_hbm.at[pl.ds(base * CODESZ, CH * CODESZ)]
        pltpu.make_async_copy(wbuf_v.at[0], tail, osem0).wait()
        pltpu.make_async_copy(wbuf_v.at[1], tail, osem1).wait()

    return decompress


def _matmul(x, w, bm=2048, bn=2048, bk=512):
    M, K = x.shape
    N = w.shape[0]
    bm, bn, bk = min(bm, M), min(bn, N), min(bk, K)

    def body(x_ref, w_ref, o_ref):
        @pl.when(pl.program_id(2) == 0)
        def _init():
            o_ref[...] = jnp.zeros_like(o_ref)

        xb = x_ref[...].astype(jnp.bfloat16)
        wb = w_ref[...].astype(jnp.bfloat16)
        o_ref[...] += lax.dot_general(
            xb, wb, (((1,), (1,)), ((), ())),
            preferred_element_type=jnp.float32)

    return pl.pallas_call(
        body,
        grid=(M // bm, N // bn, K // bk),
        in_specs=[
            pl.BlockSpec((bm, bk), lambda m, n, k: (m, k)),
            pl.BlockSpec((bn, bk), lambda m, n, k: (n, k)),
        ],
        out_specs=pl.BlockSpec((bm, bn), lambda m, n, k: (m, n)),
        out_shape=jax.ShapeDtypeStruct((M, N), jnp.float32),
        compiler_params=pltpu.CompilerParams(
            dimension_semantics=("parallel", "parallel", "arbitrary")),
    )(x, w)


def kernel(input, Qidxs, grid):
    M = Qidxs.shape[0]
    grid2 = _widen_grid(grid)
    NB = 8
    mb = M // NB
    dec = _make_decompress(mb * Qidxs.shape[1])
    ws = []
    for i in range(NB):
        qb = Qidxs[i * mb:(i + 1) * mb].reshape(-1)
        ws.append(dec(qb, grid2).reshape(mb, -1))
    outs = [_matmul(input, wb) for wb in ws]
    return jnp.concatenate(outs, axis=1)


# speedup vs baseline: 1.1721x; 1.1721x over previous
import functools

import jax
import jax.numpy as jnp
from jax import lax
from jax.experimental import pallas as pl
from jax.experimental.pallas import tpu as pltpu
from jax.experimental.pallas import tpu_sc as plsc

RESID_SCALE = 1.0 / 3.45
CODESZ = 8


def _sc_dims():
    try:
        info = plsc.get_sparse_core_info()
        return info.num_cores, info.num_subcores, info.num_lanes
    except (RuntimeError, ValueError):
        return 2, 16, 16


def _widen_grid(grid, br=8192):
    G, D = grid.shape

    def body(g_ref, o_ref):
        g = g_ref[...]
        o_ref[...] = jnp.concatenate([g, g], axis=1)

    return pl.pallas_call(
        body,
        grid=(G // br,),
        in_specs=[pl.BlockSpec((br, D), lambda i: (i, 0))],
        out_specs=pl.BlockSpec((br, 2 * D), lambda i: (i, 0)),
        out_shape=jax.ShapeDtypeStruct((G, 2 * D), jnp.float32),
    )(grid)


def _make_decompress(n_packed):
    NC, NS, L = _sc_dims()
    NW = NC * NS
    assert n_packed % NW == 0
    per_w = n_packed // NW
    CH = 1024
    assert per_w % (2 * CH) == 0
    n_ch = per_w // CH
    STREAM = 128
    n_streams = CH // STREAM
    mesh = plsc.VectorSubcoreMesh(core_axis_name="c", subcore_axis_name="s",
                                  num_cores=NC, num_subcores=NS)

    @functools.partial(
        pl.kernel,
        out_type=jax.ShapeDtypeStruct((n_packed * CODESZ,), jnp.float32),
        mesh=mesh,
        scratch_types=[
            pltpu.VMEM((2, CH), jnp.int32),
            pltpu.VMEM((2, CH), jnp.int32),
            pltpu.VMEM((2, CH), jnp.int32),
            pltpu.VMEM((2, CH, 2 * CODESZ), jnp.float32),
            pltpu.VMEM((2, CH, 2 * CODESZ), jnp.float32),
            pltpu.VMEM((2, CH * CODESZ), jnp.float32),
            pltpu.SemaphoreType.DMA,
            pltpu.SemaphoreType.DMA,
            pltpu.SemaphoreType.DMA,
            pltpu.SemaphoreType.DMA,
        ],
        compiler_params=pltpu.CompilerParams(use_tc_tiling_on_sc=False,
                                             vmem_limit_bytes=4 * 1024 * 1024),
    )
    def decompress(q_hbm, grid2_hbm, w_hbm, packed_v, hi_v, lo_v, ga_v, gb_v,
                   wbuf_v, gsem0, gsem1, osem0, osem1):
        wid = lax.axis_index("s") * NC + lax.axis_index("c")
        base = wid * per_w
        lane = lax.iota(jnp.int32, L)
        lmask = lane < CODESZ
        sixteen = jnp.full((L,), 16, jnp.int32)
        mask16 = jnp.full((L,), 0xFFFF, jnp.int32)
        scale_v = jnp.full((L,), RESID_SCALE, jnp.float32)

        def fire(c, b, gsem):
            row0 = base + c * CH
            pltpu.sync_copy(q_hbm.at[pl.ds(row0, CH)], packed_v.at[b])

            def split_body(i, _):
                v = packed_v[b, pl.ds(i * L, L)]
                hi_v[b, pl.ds(i * L, L)] = lax.shift_right_logical(v, sixteen)
                lo_v[b, pl.ds(i * L, L)] = lax.bitwise_and(v, mask16)
                return 0

            lax.fori_loop(0, CH // L, split_body, 0, unroll=4)
            for j in range(n_streams):
                sl = pl.ds(j * STREAM, STREAM)
                pltpu.async_copy(
                    grid2_hbm.at[hi_v.at[b, sl]], ga_v.at[b, sl], gsem)
                pltpu.async_copy(
                    grid2_hbm.at[lo_v.at[b, sl]], gb_v.at[b, sl], gsem)

        def wait_gathers(b, gsem):
            dummy = grid2_hbm.at[pl.ds(0, CH)]
            pltpu.make_async_copy(dummy, ga_v.at[b], gsem).wait()
            pltpu.make_async_copy(dummy, gb_v.at[b], gsem).wait()

        def fma_store(c, b, osem):
            @pl.when(c >= 2)
            def _wait_prev_store():
                pltpu.make_async_copy(
                    wbuf_v.at[b], w_hbm.at[pl.ds(base * CODESZ, CH * CODESZ)],
                    osem).wait()

            @plsc.parallel_loop(0, CH // 2, unroll=16)
            def fma_body(i):
                a0 = ga_v[b, 2 * i, :]
                a1 = ga_v[b, 2 * i + 1, :]
                b0 = gb_v[b, 2 * i, :]
                b1 = gb_v[b, 2 * i + 1, :]
                wa = jnp.where(lmask, a0, a1)
                wb = jnp.where(lmask, b0, b1)
                wbuf_v[b, pl.ds(i * L, L)] = wa + wb * scale_v

            row0 = base + c * CH
            pltpu.async_copy(
                wbuf_v.at[b], w_hbm.at[pl.ds(row0 * CODESZ, CH * CODESZ)], osem)

        fire(0, 0, gsem0)

        def loop_body(c2, carry):
            c = 2 * c2

            @pl.when(c + 1 < n_ch)
            def _f1():
                fire(c + 1, 1, gsem1)

            wait_gathers(0, gsem0)
            fma_store(c, 0, osem0)

            @pl.when(c + 2 < n_ch)
            def _f2():
                fire(c + 2, 0, gsem0)

            wait_gathers(1, gsem1)
            fma_store(c + 1, 1, osem1)
            return carry

        lax.fori_loop(0, n_ch // 2, loop_body, 0)
        tail = w_hbm.at[pl.ds(base * CODESZ, CH * CODESZ)]
        pltpu.make_async_copy(wbuf_v.at[0], tail, osem0).wait()
        pltpu.make_async_copy(wbuf_v.at[1], tail, osem1).wait()

    return decompress


def _matmul(x, w, bm=2048, bn=2048, bk=512):
    M, K = x.shape
    N = w.shape[0]
    bm, bn, bk = min(bm, M), min(bn, N), min(bk, K)

    def body(x_ref, w_ref, o_ref):
        @pl.when(pl.program_id(2) == 0)
        def _init():
            o_ref[...] = jnp.zeros_like(o_ref)

        xb = x_ref[...].astype(jnp.bfloat16)
        wb = w_ref[...].astype(jnp.bfloat16)
        o_ref[...] += lax.dot_general(
            xb, wb, (((1,), (1,)), ((), ())),
            preferred_element_type=jnp.float32)

    return pl.pallas_call(
        body,
        grid=(M // bm, N // bn, K // bk),
        in_specs=[
            pl.BlockSpec((bm, bk), lambda m, n, k: (m, k)),
            pl.BlockSpec((bn, bk), lambda m, n, k: (n, k)),
        ],
        out_specs=pl.BlockSpec((bm, bn), lambda m, n, k: (m, n)),
        out_shape=jax.ShapeDtypeStruct((M, N), jnp.float32),
        compiler_params=pltpu.CompilerParams(
            dimension_semantics=("parallel", "parallel", "arbitrary")),
    )(x, w)


def kernel(input, Qidxs, grid):
    M = Qidxs.shape[0]
    grid2 = _widen_grid(grid)
    NB = 4
    mb = M // NB
    dec = _make_decompress(mb * Qidxs.shape[1])
    ws = []
    for i in range(NB):
        qb = Qidxs[i * mb:(i + 1) * mb].reshape(-1)
        ws.append(dec(qb, grid2).reshape(mb, -1))
    outs = [_matmul(input, wb) for wb in ws]
    return jnp.concatenate(outs, axis=1)


# speedup vs baseline: 1.1891x; 1.0145x over previous
import functools

import jax
import jax.numpy as jnp
from jax import lax
from jax.experimental import pallas as pl
from jax.experimental.pallas import tpu as pltpu
from jax.experimental.pallas import tpu_sc as plsc

RESID_SCALE = 1.0 / 3.45
CODESZ = 8


def _sc_dims():
    try:
        info = plsc.get_sparse_core_info()
        return info.num_cores, info.num_subcores, info.num_lanes
    except (RuntimeError, ValueError):
        return 2, 16, 16


def _widen_grid(grid, br=8192):
    G, D = grid.shape

    def body(g_ref, o_ref):
        g = g_ref[...]
        o_ref[...] = jnp.concatenate([g, g], axis=1)

    return pl.pallas_call(
        body,
        grid=(G // br,),
        in_specs=[pl.BlockSpec((br, D), lambda i: (i, 0))],
        out_specs=pl.BlockSpec((br, 2 * D), lambda i: (i, 0)),
        out_shape=jax.ShapeDtypeStruct((G, 2 * D), jnp.float32),
    )(grid)


def _make_decompress(n_packed):
    NC, NS, L = _sc_dims()
    NW = NC * NS
    assert n_packed % NW == 0
    per_w = n_packed // NW
    CH = 1024
    assert per_w % (2 * CH) == 0
    n_ch = per_w // CH
    STREAM = 128
    n_streams = CH // STREAM
    mesh = plsc.VectorSubcoreMesh(core_axis_name="c", subcore_axis_name="s",
                                  num_cores=NC, num_subcores=NS)

    @functools.partial(
        pl.kernel,
        out_type=jax.ShapeDtypeStruct((n_packed * CODESZ,), jnp.float32),
        mesh=mesh,
        scratch_types=[
            pltpu.VMEM((2, CH), jnp.int32),
            pltpu.VMEM((2, CH), jnp.int32),
            pltpu.VMEM((2, CH), jnp.int32),
            pltpu.VMEM((2, CH, 2 * CODESZ), jnp.float32),
            pltpu.VMEM((2, CH, 2 * CODESZ), jnp.float32),
            pltpu.VMEM((2, CH * CODESZ), jnp.float32),
            pltpu.SemaphoreType.DMA,
            pltpu.SemaphoreType.DMA,
            pltpu.SemaphoreType.DMA,
            pltpu.SemaphoreType.DMA,
        ],
        compiler_params=pltpu.CompilerParams(use_tc_tiling_on_sc=False,
                                             vmem_limit_bytes=4 * 1024 * 1024),
    )
    def decompress(q_hbm, grid2_hbm, w_hbm, packed_v, hi_v, lo_v, ga_v, gb_v,
                   wbuf_v, gsem0, gsem1, osem0, osem1):
        wid = lax.axis_index("s") * NC + lax.axis_index("c")
        base = wid * per_w
        lane = lax.iota(jnp.int32, L)
        lmask = lane < CODESZ
        sixteen = jnp.full((L,), 16, jnp.int32)
        mask16 = jnp.full((L,), 0xFFFF, jnp.int32)
        scale_v = jnp.full((L,), RESID_SCALE, jnp.float32)

        def fire(c, b, gsem):
            row0 = base + c * CH
            pltpu.sync_copy(q_hbm.at[pl.ds(row0, CH)], packed_v.at[b])

            def split_body(i, _):
                v = packed_v[b, pl.ds(i * L, L)]
                hi_v[b, pl.ds(i * L, L)] = lax.shift_right_logical(v, sixteen)
                lo_v[b, pl.ds(i * L, L)] = lax.bitwise_and(v, mask16)
                return 0

            lax.fori_loop(0, CH // L, split_body, 0, unroll=4)
            for j in range(n_streams):
                sl = pl.ds(j * STREAM, STREAM)
                pltpu.async_copy(
                    grid2_hbm.at[hi_v.at[b, sl]], ga_v.at[b, sl], gsem)
                pltpu.async_copy(
                    grid2_hbm.at[lo_v.at[b, sl]], gb_v.at[b, sl], gsem)

        def wait_gathers(b, gsem):
            dummy = grid2_hbm.at[pl.ds(0, CH)]
            pltpu.make_async_copy(dummy, ga_v.at[b], gsem).wait()
            pltpu.make_async_copy(dummy, gb_v.at[b], gsem).wait()

        def fma_store(c, b, osem):
            @pl.when(c >= 2)
            def _wait_prev_store():
                pltpu.make_async_copy(
                    wbuf_v.at[b], w_hbm.at[pl.ds(base * CODESZ, CH * CODESZ)],
                    osem).wait()

            @plsc.parallel_loop(0, CH // 2, unroll=16)
            def fma_body(i):
                a0 = ga_v[b, 2 * i, :]
                a1 = ga_v[b, 2 * i + 1, :]
                b0 = gb_v[b, 2 * i, :]
                b1 = gb_v[b, 2 * i + 1, :]
                wa = jnp.where(lmask, a0, a1)
                wb = jnp.where(lmask, b0, b1)
                wbuf_v[b, pl.ds(i * L, L)] = wa + wb * scale_v

            row0 = base + c * CH
            pltpu.async_copy(
                wbuf_v.at[b], w_hbm.at[pl.ds(row0 * CODESZ, CH * CODESZ)], osem)

        fire(0, 0, gsem0)

        def loop_body(c2, carry):
            c = 2 * c2

            @pl.when(c + 1 < n_ch)
            def _f1():
                fire(c + 1, 1, gsem1)

            wait_gathers(0, gsem0)
            fma_store(c, 0, osem0)

            @pl.when(c + 2 < n_ch)
            def _f2():
                fire(c + 2, 0, gsem0)

            wait_gathers(1, gsem1)
            fma_store(c + 1, 1, osem1)
            return carry

        lax.fori_loop(0, n_ch // 2, loop_body, 0)
        tail = w_hbm.at[pl.ds(base * CODESZ, CH * CODESZ)]
        pltpu.make_async_copy(wbuf_v.at[0], tail, osem0).wait()
        pltpu.make_async_copy(wbuf_v.at[1], tail, osem1).wait()

    return decompress


def _matmul(x, w, bm=2048, bn=2048, bk=1024):
    M, K = x.shape
    N = w.shape[0]
    bm, bn, bk = min(bm, M), min(bn, N), min(bk, K)

    def body(x_ref, w_ref, o_ref):
        @pl.when(pl.program_id(2) == 0)
        def _init():
            o_ref[...] = jnp.zeros_like(o_ref)

        xb = x_ref[...].astype(jnp.bfloat16)
        wb = w_ref[...].astype(jnp.bfloat16)
        o_ref[...] += lax.dot_general(
            xb, wb, (((1,), (1,)), ((), ())),
            preferred_element_type=jnp.float32)

    return pl.pallas_call(
        body,
        grid=(M // bm, N // bn, K // bk),
        in_specs=[
            pl.BlockSpec((bm, bk), lambda m, n, k: (m, k)),
            pl.BlockSpec((bn, bk), lambda m, n, k: (n, k)),
        ],
        out_specs=pl.BlockSpec((bm, bn), lambda m, n, k: (m, n)),
        out_shape=jax.ShapeDtypeStruct((M, N), jnp.float32),
        compiler_params=pltpu.CompilerParams(
            dimension_semantics=("parallel", "parallel", "arbitrary")),
    )(x, w)


def kernel(input, Qidxs, grid):
    M = Qidxs.shape[0]
    grid2 = _widen_grid(grid)
    NB = 4
    mb = M // NB
    dec = _make_decompress(mb * Qidxs.shape[1])
    ws = []
    for i in range(NB):
        qb = Qidxs[i * mb:(i + 1) * mb].reshape(-1)
        ws.append(dec(qb, grid2).reshape(mb, -1))
    outs = [_matmul(input, wb) for wb in ws]
    return jnp.concatenate(outs, axis=1)
